# baseline (speedup 1.0000x reference)
import jax
import jax.numpy as jnp
from jax.experimental import pallas as pl


def _gat_layer(x, src2, dst2, ew2, Wl, bl, Wr, br, We, att, bias):
    n = x.shape[0]
    xl = x @ Wl + bl
    xr = x @ Wr + br
    m = xl[src2] + xr[dst2] + ew2 @ We
    m = jax.nn.leaky_relu(m, negative_slope=0.2)
    alpha = m @ att
    amax = jax.ops.segment_max(alpha, dst2, num_segments=n)
    ex = jnp.exp(alpha - amax[dst2])
    den = jax.ops.segment_sum(ex, dst2, num_segments=n)
    a = ex / (den[dst2] + 1e-16)
    out = jax.ops.segment_sum(xl[src2] * a[:, None], dst2, num_segments=n)
    return out + bias


def _final_matmul_kernel(h_ref, w_ref, b_ref, o_ref):
    o_ref[...] = h_ref[...] @ w_ref[...] + b_ref[...]


def kernel(x, edge_index, edge_attr, Wl1, bl1, Wr1, br1, We1, att1, b1, Wl2, bl2, Wr2, br2, We2, att2, b2, Wl3, bl3, Wr3, br3, We3, att3, b3, Wout, bout):
    n = x.shape[0]
    src, dst = edge_index[0], edge_index[1]
    safe = jnp.where(edge_attr == 0, 1.0, edge_attr)
    ew = jnp.where(edge_attr == 0, 1.0 / 0.0001, 1.0 / safe)
    sl = jnp.arange(n, dtype=src.dtype)
    src2 = jnp.concatenate([src, sl])
    dst2 = jnp.concatenate([dst, sl])
    ew2 = jnp.concatenate([ew, jnp.zeros((n, ew.shape[1]), ew.dtype)], axis=0)
    h = _gat_layer(x, src2, dst2, ew2, Wl1, bl1, Wr1, br1, We1, att1, b1)
    h = jax.nn.relu(h)
    h = _gat_layer(h, src2, dst2, ew2, Wl2, bl2, Wr2, br2, We2, att2, b2)
    h = jax.nn.relu(h)
    h = _gat_layer(h, src2, dst2, ew2, Wl3, bl3, Wr3, br3, We3, att3, b3)
    out = pl.pallas_call(
        _final_matmul_kernel,
        out_shape=jax.ShapeDtypeStruct((n, 1), h.dtype),
    )(h, Wout, bout)
    return out.squeeze(-1)


# speedup vs baseline: 5.1782x; 5.1782x over previous
import dataclasses
import functools

import jax
import jax.numpy as jnp
from jax import lax
from jax.experimental import pallas as pl
from jax.experimental.pallas import tpu as pltpu
from jax.experimental.pallas import tpu_sc as plsc

N = 10000
E = 320000
H = 128
P = 10240
PR = P // 128
NTILES = 32
EPT = E // NTILES
CH = 80
NCH = EPT // CH
NEG = -1.0e30

_mesh = plsc.VectorSubcoreMesh(core_axis_name="c", subcore_axis_name="s")

_SC_PARAMS = pltpu.CompilerParams()
if "needs_layout_passes" in pltpu.CompilerParams.__dataclass_fields__:
    _SC_PARAMS = dataclasses.replace(_SC_PARAMS, needs_layout_passes=False)

_GATHER_DNUMS = lax.GatherDimensionNumbers(
    offset_dims=(), collapsed_slice_dims=(0,), start_index_map=(0,))


def _lane_gather(v, idx):
    return lax.gather(v, idx[:, None], _GATHER_DNUMS, slice_sizes=(1,),
                      mode=lax.GatherScatterMode.PROMISE_IN_BOUNDS)


def _bf16_round(v):
    u = plsc.bitcast(v, jnp.uint32)
    r = lax.bitwise_and(lax.shift_right_logical(u, jnp.uint32(16)),
                        jnp.uint32(1))
    u = u + jnp.uint32(0x7FFF) + r
    u = lax.bitwise_and(u, jnp.uint32(0xFFFF0000))
    return plsc.bitcast(u, jnp.float32)


def _ew_kernel(ea_ref, o_ref):
    ea = ea_ref[...]
    safe = jnp.where(ea == 0.0, 1.0, ea)
    o_ref[...] = jnp.where(ea == 0.0, 1.0 / 0.0001, 1.0 / safe)


def _ew_transform(edge_attr):
    ea = edge_attr.reshape(E // 128, 128)
    out = pl.pallas_call(
        _ew_kernel,
        out_shape=jax.ShapeDtypeStruct((E // 128, 128), jnp.float32),
    )(ea)
    return out.reshape(E)


def _proj_kernel(x_ref, wl_ref, bl_ref, wr_ref, br_ref, att_ref, xl_ref,
                 xr_ref, sa_ref):
    xb = x_ref[...].astype(jnp.bfloat16)
    xl = jnp.dot(xb, wl_ref[...].astype(jnp.bfloat16),
                 preferred_element_type=jnp.float32) + bl_ref[...]
    xr = jnp.dot(xb, wr_ref[...].astype(jnp.bfloat16),
                 preferred_element_type=jnp.float32) + br_ref[...]
    xl_ref[...] = xl
    xr_ref[...] = xr
    m = xl + xr
    m = jnp.where(m >= 0.0, m, 0.2 * m)
    sa_ref[...] = jnp.dot(m.astype(jnp.bfloat16),
                          att_ref[...].astype(jnp.bfloat16),
                          preferred_element_type=jnp.float32)


def _proj(x, Wl, bl, Wr, br, att):
    grid = (P // 1024,)
    return pl.pallas_call(
        _proj_kernel,
        grid=grid,
        in_specs=[
            pl.BlockSpec((1024, H), lambda i: (i, 0)),
            pl.BlockSpec((H, H), lambda i: (0, 0)),
            pl.BlockSpec((1, H), lambda i: (0, 0)),
            pl.BlockSpec((H, H), lambda i: (0, 0)),
            pl.BlockSpec((1, H), lambda i: (0, 0)),
            pl.BlockSpec((H, 1), lambda i: (0, 0)),
        ],
        out_specs=[
            pl.BlockSpec((1024, H), lambda i: (i, 0)),
            pl.BlockSpec((1024, H), lambda i: (i, 0)),
            pl.BlockSpec((1024, 1), lambda i: (i, 0)),
        ],
        out_shape=[
            jax.ShapeDtypeStruct((P, H), jnp.float32),
            jax.ShapeDtypeStruct((P, H), jnp.float32),
            jax.ShapeDtypeStruct((P, 1), jnp.float32),
        ],
    )(x, Wl, bl.reshape(1, H), Wr, br.reshape(1, H), att.reshape(H, 1))


def _sc_gsum_body(xl_hbm, xr_hbm, src_hbm, dst_hbm, msum_hbm,
                  src_v, dst_v, xlr_v, xrr_v):
    c = lax.axis_index("c")
    s = lax.axis_index("s")
    wid = s * 2 + c
    base = wid * EPT

    @pl.loop(0, NCH)
    def _(ci):
        off = base + ci * CH
        pltpu.sync_copy(src_hbm.at[pl.ds(off, CH)], src_v)
        pltpu.sync_copy(dst_hbm.at[pl.ds(off, CH)], dst_v)
        pltpu.sync_copy(xl_hbm.at[src_v], xlr_v)
        pltpu.sync_copy(xr_hbm.at[dst_v], xrr_v)

        @pl.loop(0, CH)
        def _(e):
            for kk in range(8):
                xlr_v[e, pl.ds(kk * 16, 16)] = (
                    xlr_v[e, pl.ds(kk * 16, 16)] + xrr_v[e, pl.ds(kk * 16, 16)])

        pltpu.sync_copy(xlr_v, msum_hbm.at[pl.ds(off, CH)])


def _sc_gsum(xl, xr, src, dst):
    kfn = pl.kernel(
        _sc_gsum_body,
        out_type=jax.ShapeDtypeStruct((E, H), jnp.float32),
        mesh=_mesh,
        compiler_params=_SC_PARAMS,
        scratch_types=[
            pltpu.VMEM((CH,), jnp.int32),
            pltpu.VMEM((CH,), jnp.int32),
            pltpu.VMEM((CH, H), jnp.float32),
            pltpu.VMEM((CH, H), jnp.float32),
        ],
    )
    return kfn(xl, xr, src, dst)


EB = 2000


def _alpha_kernel(ms_ref, ew_ref, we_ref, att_ref, o_ref):
    m = ms_ref[...] + ew_ref[...] * we_ref[...]
    m = jnp.where(m >= 0.0, m, 0.2 * m)
    mb = m.astype(jnp.bfloat16)
    o_ref[...] = jnp.dot(mb, att_ref[...].astype(jnp.bfloat16),
                         preferred_element_type=jnp.float32)


def _alpha_tc(msum, ew, We, att):
    grid = (E // EB,)
    return pl.pallas_call(
        _alpha_kernel,
        grid=grid,
        in_specs=[
            pl.BlockSpec((EB, H), lambda i: (i, 0)),
            pl.BlockSpec((EB, 1), lambda i: (i, 0)),
            pl.BlockSpec((1, H), lambda i: (0, 0)),
            pl.BlockSpec((H, 1), lambda i: (0, 0)),
        ],
        out_specs=pl.BlockSpec((EB, 1), lambda i: (i, 0)),
        out_shape=jax.ShapeDtypeStruct((E, 1), jnp.float32),
    )(msum, ew.reshape(E, 1), We.reshape(1, H), att.reshape(H, 1))


def _sc_segmax_body(dst_hbm, alpha_hbm, amax_part_hbm, dst_v, alpha_v, amax_v):
    c = lax.axis_index("c")
    s = lax.axis_index("s")
    wid = s * 2 + c
    base = wid * EPT

    neg16 = jnp.full((16,), NEG, jnp.float32)

    @pl.loop(0, PR)
    def _(i):
        @pl.loop(0, 128, step=16)
        def _(j):
            amax_v[i, pl.ds(j, 16)] = neg16

    lane = jnp.arange(16, dtype=jnp.int32)
    rot1 = (lane + 1) & 15

    @pl.loop(0, NCH)
    def _(ci):
        off = base + ci * CH
        pltpu.sync_copy(dst_hbm.at[pl.ds(off, CH)], dst_v)
        pltpu.sync_copy(alpha_hbm.at[pl.ds(off, CH)], alpha_v)

        @pl.loop(0, CH // 16)
        def _(g):
            dst16 = dst_v[pl.ds(g * 16, 16)]
            aph = alpha_v[pl.ds(g * 16, 16)]
            kr = dst16
            vr = aph
            vmax = aph
            for _r in range(15):
                kr = _lane_gather(kr, rot1)
                vr = _lane_gather(vr, rot1)
                vmax = jnp.where(kr == dst16, jnp.maximum(vmax, vr), vmax)
            i0 = lax.shift_right_logical(dst16, 7)
            i1 = lax.bitwise_and(dst16, 127)
            cur = plsc.load_gather(amax_v, [i0, i1])
            plsc.store_scatter(amax_v, [i0, i1], jnp.maximum(cur, vmax))

    pltpu.sync_copy(amax_v, amax_part_hbm.at[wid])


def _sc_segmax(dst, alpha):
    kfn = pl.kernel(
        _sc_segmax_body,
        out_type=jax.ShapeDtypeStruct((NTILES, PR, 128), jnp.float32),
        mesh=_mesh,
        compiler_params=_SC_PARAMS,
        scratch_types=[
            pltpu.VMEM((CH,), jnp.int32),
            pltpu.VMEM((CH,), jnp.float32),
            pltpu.VMEM((PR, 128), jnp.float32),
        ],
    )
    return kfn(dst, alpha)


def _amax_kernel(ap_ref, sa_ref, o_ref):
    o_ref[...] = jnp.maximum(jnp.max(ap_ref[...], axis=0), sa_ref[...])


def _amax_combine(amax_part, salpha):
    return pl.pallas_call(
        _amax_kernel,
        out_shape=jax.ShapeDtypeStruct((PR, 128), jnp.float32),
    )(amax_part, salpha.reshape(PR, 128))


def _sc_pass_b_body(xl_hbm, src_hbm, dst_hbm, alpha_hbm, amaxf_hbm, zeros_hbm,
                    den_part_hbm, acc_part_hbm,
                    src_v, dst_v, alpha_v, amax_v, den_v, rows_v, acc_sh):
    c = lax.axis_index("c")
    s = lax.axis_index("s")
    wid = s * 2 + c
    base = wid * EPT

    pltpu.sync_copy(amaxf_hbm, amax_v)

    z16 = jnp.zeros((16,), jnp.float32)

    @pl.loop(0, PR)
    def _(i):
        @pl.loop(0, 128, step=16)
        def _(j):
            den_v[i, pl.ds(j, 16)] = z16

    pltpu.sync_copy(zeros_hbm, acc_sh.at[pl.ds(s * (P // 16), P // 16)])
    plsc.subcore_barrier()

    lane = jnp.arange(16, dtype=jnp.int32)
    rot1 = (lane + 1) & 15

    @pl.loop(0, NCH)
    def _(ci):
        off = base + ci * CH
        pltpu.sync_copy(src_hbm.at[pl.ds(off, CH)], src_v)
        pltpu.sync_copy(dst_hbm.at[pl.ds(off, CH)], dst_v)
        pltpu.sync_copy(alpha_hbm.at[pl.ds(off, CH)], alpha_v)
        pltpu.sync_copy(xl_hbm.at[src_v], rows_v)

        @pl.loop(0, CH // 16)
        def _(g):
            dst16 = dst_v[pl.ds(g * 16, 16)]
            a16 = alpha_v[pl.ds(g * 16, 16)]
            i0 = lax.shift_right_logical(dst16, 7)
            i1 = lax.bitwise_and(dst16, 127)
            av = plsc.load_gather(amax_v, [i0, i1])
            exv = jnp.exp(a16 - av)

            kr = dst16
            vr = exv
            vsum = exv
            for _r in range(15):
                kr = _lane_gather(kr, rot1)
                vr = _lane_gather(vr, rot1)
                vsum = jnp.where(kr == dst16, vsum + vr, vsum)
            cur = plsc.load_gather(den_v, [i0, i1])
            plsc.store_scatter(den_v, [i0, i1], cur + vsum)

            for l in range(16):
                e = g * 16 + l
                b = _lane_gather(exv, jnp.full((16,), l, jnp.int32))
                for kk in range(8):
                    rows_v[e, pl.ds(kk * 16, 16)] = (
                        rows_v[e, pl.ds(kk * 16, 16)] * b)

        pltpu.sync_copy(rows_v, acc_sh.at[dst_v], add=True)

    plsc.subcore_barrier()
    pltpu.sync_copy(acc_sh.at[pl.ds(s * (P // 16), P // 16)],
                    acc_part_hbm.at[c, pl.ds(s * (P // 16), P // 16)])
    pltpu.sync_copy(den_v, den_part_hbm.at[wid])


def _sc_pass_b(xl, src, dst, alpha, amaxf, zeros_block):
    kfn = pl.kernel(
        _sc_pass_b_body,
        out_type=[
            jax.ShapeDtypeStruct((NTILES, PR, 128), jnp.float32),
            jax.ShapeDtypeStruct((2, P, H), jnp.float32),
        ],
        mesh=_mesh,
        compiler_params=_SC_PARAMS,
        scratch_types=[
            pltpu.VMEM((CH,), jnp.int32),
            pltpu.VMEM((CH,), jnp.int32),
            pltpu.VMEM((CH,), jnp.float32),
            pltpu.VMEM((PR, 128), jnp.float32),
            pltpu.VMEM((PR, 128), jnp.float32),
            pltpu.VMEM((CH, H), jnp.float32),
            pltpu.VMEM_SHARED((P, H), jnp.float32),
        ],
    )
    return kfn(xl, src, dst, alpha, amaxf, zeros_block)


def _densum_kernel(dp_ref, o_ref):
    o_ref[...] = jnp.sum(dp_ref[...], axis=0)


def _den_combine(den_part):
    return pl.pallas_call(
        _densum_kernel,
        out_shape=jax.ShapeDtypeStruct((PR, 128), jnp.float32),
    )(den_part)


def _finalize_kernel(acc_ref, dr_ref, sa_ref, am_ref, xl_ref, b_ref, o_ref, *,
                     relu):
    exs = jnp.exp(sa_ref[...] - am_ref[...])
    den = dr_ref[...] + exs
    r = 1.0 / (den + 1e-16)
    h = (acc_ref[0] + acc_ref[1] + exs * xl_ref[...]) * r + b_ref[...]
    if relu:
        h = jnp.maximum(h, 0.0)
    o_ref[...] = h


def _finalize(acc_part, den_red, salpha, amaxf, xl, bias, relu):
    grid = (P // 1024,)
    return pl.pallas_call(
        functools.partial(_finalize_kernel, relu=relu),
        grid=grid,
        in_specs=[
            pl.BlockSpec((2, 1024, H), lambda i: (0, i, 0)),
            pl.BlockSpec((1024, 1), lambda i: (i, 0)),
            pl.BlockSpec((1024, 1), lambda i: (i, 0)),
            pl.BlockSpec((1024, 1), lambda i: (i, 0)),
            pl.BlockSpec((1024, H), lambda i: (i, 0)),
            pl.BlockSpec((1, H), lambda i: (0, 0)),
        ],
        out_specs=pl.BlockSpec((1024, H), lambda i: (i, 0)),
        out_shape=jax.ShapeDtypeStruct((P, H), jnp.float32),
    )(acc_part, den_red, salpha, amaxf, xl, bias.reshape(1, H))


def _outproj_kernel(h_ref, w_ref, b_ref, o_ref):
    o_ref[...] = jnp.dot(h_ref[...].astype(jnp.bfloat16),
                         w_ref[...].astype(jnp.bfloat16),
                         preferred_element_type=jnp.float32) + b_ref[0, 0]


def _outproj(h, Wout, bout):
    grid = (P // 1024,)
    return pl.pallas_call(
        _outproj_kernel,
        grid=grid,
        in_specs=[
            pl.BlockSpec((1024, H), lambda i: (i, 0)),
            pl.BlockSpec((H, 1), lambda i: (0, 0)),
            pl.BlockSpec((1, 1), lambda i: (0, 0)),
        ],
        out_specs=pl.BlockSpec((1024, 1), lambda i: (i, 0)),
        out_shape=jax.ShapeDtypeStruct((P, 1), jnp.float32),
    )(h, Wout.reshape(H, 1), bout.reshape(1, 1))


def _gat_layer_sc(x, src, dst, ew, zeros_block, Wl, bl, Wr, br, We, att, bias,
                  relu):
    xl, xr, salpha = _proj(x, Wl, bl, Wr, br, att)
    msum = _sc_gsum(xl, xr, src, dst)
    alpha = _alpha_tc(msum, ew, We, att).reshape(E)
    amax_part = _sc_segmax(dst, alpha)
    amaxf = _amax_combine(amax_part, salpha)
    den_part, acc_part = _sc_pass_b(xl, src, dst, alpha, amaxf, zeros_block)
    den_red = _den_combine(den_part)
    return _finalize(acc_part, den_red.reshape(P, 1), salpha,
                     amaxf.reshape(P, 1), xl, bias, relu)


def kernel(x, edge_index, edge_attr, Wl1, bl1, Wr1, br1, We1, att1, b1, Wl2,
           bl2, Wr2, br2, We2, att2, b2, Wl3, bl3, Wr3, br3, We3, att3, b3,
           Wout, bout):
    src = edge_index[0]
    dst = edge_index[1]
    ew = _ew_transform(edge_attr.reshape(E))
    x_pad = jnp.zeros((P, x.shape[1]), jnp.float32).at[:N].set(x)
    zeros_block = jnp.zeros((P // 16, H), jnp.float32)

    h = _gat_layer_sc(x_pad, src, dst, ew, zeros_block, Wl1, bl1, Wr1, br1,
                      We1.reshape(H), att1, b1, relu=True)
    h = _gat_layer_sc(h, src, dst, ew, zeros_block, Wl2, bl2, Wr2, br2,
                      We2.reshape(H), att2, b2, relu=True)
    h = _gat_layer_sc(h, src, dst, ew, zeros_block, Wl3, bl3, Wr3, br3,
                      We3.reshape(H), att3, b3, relu=False)
    out = _outproj(h, Wout, bout)
    return out.reshape(P)[:N]


# speedup vs baseline: 5.9978x; 1.1583x over previous
import dataclasses
import functools

import jax
import jax.numpy as jnp
from jax import lax
from jax.experimental import pallas as pl
from jax.experimental.pallas import tpu as pltpu
from jax.experimental.pallas import tpu_sc as plsc

N = 10000
E = 320000
H = 128
P = 10240
PR = P // 128
NTILES = 32
EPT = E // NTILES
CH = 80
NCH = EPT // CH
NEG = -1.0e30

_mesh = plsc.VectorSubcoreMesh(core_axis_name="c", subcore_axis_name="s")

_SC_PARAMS = pltpu.CompilerParams()
if "needs_layout_passes" in pltpu.CompilerParams.__dataclass_fields__:
    _SC_PARAMS = dataclasses.replace(_SC_PARAMS, needs_layout_passes=False)

_GATHER_DNUMS = lax.GatherDimensionNumbers(
    offset_dims=(), collapsed_slice_dims=(0,), start_index_map=(0,))


def _lane_gather(v, idx):
    return lax.gather(v, idx[:, None], _GATHER_DNUMS, slice_sizes=(1,),
                      mode=lax.GatherScatterMode.PROMISE_IN_BOUNDS)


def _bf16_round(v):
    u = plsc.bitcast(v, jnp.uint32)
    r = lax.bitwise_and(lax.shift_right_logical(u, jnp.uint32(16)),
                        jnp.uint32(1))
    u = u + jnp.uint32(0x7FFF) + r
    u = lax.bitwise_and(u, jnp.uint32(0xFFFF0000))
    return plsc.bitcast(u, jnp.float32)


def _ew_kernel(ea_ref, o_ref):
    ea = ea_ref[...]
    safe = jnp.where(ea == 0.0, 1.0, ea)
    o_ref[...] = jnp.where(ea == 0.0, 1.0 / 0.0001, 1.0 / safe)


def _ew_transform(edge_attr):
    ea = edge_attr.reshape(E // 128, 128)
    out = pl.pallas_call(
        _ew_kernel,
        out_shape=jax.ShapeDtypeStruct((E // 128, 128), jnp.float32),
    )(ea)
    return out.reshape(E)


def _proj_kernel(x_ref, wl_ref, bl_ref, wr_ref, br_ref, att_ref, xl_ref,
                 xr_ref, sa_ref):
    xb = x_ref[...].astype(jnp.bfloat16)
    xl = jnp.dot(xb, wl_ref[...].astype(jnp.bfloat16),
                 preferred_element_type=jnp.float32) + bl_ref[...]
    xr = jnp.dot(xb, wr_ref[...].astype(jnp.bfloat16),
                 preferred_element_type=jnp.float32) + br_ref[...]
    xl_ref[...] = xl
    xr_ref[...] = xr
    m = xl + xr
    m = jnp.where(m >= 0.0, m, 0.2 * m)
    sa_ref[...] = jnp.dot(m.astype(jnp.bfloat16),
                          att_ref[...].astype(jnp.bfloat16),
                          preferred_element_type=jnp.float32)


def _proj(x, Wl, bl, Wr, br, att):
    grid = (P // 1024,)
    return pl.pallas_call(
        _proj_kernel,
        grid=grid,
        in_specs=[
            pl.BlockSpec((1024, H), lambda i: (i, 0)),
            pl.BlockSpec((H, H), lambda i: (0, 0)),
            pl.BlockSpec((1, H), lambda i: (0, 0)),
            pl.BlockSpec((H, H), lambda i: (0, 0)),
            pl.BlockSpec((1, H), lambda i: (0, 0)),
            pl.BlockSpec((H, 1), lambda i: (0, 0)),
        ],
        out_specs=[
            pl.BlockSpec((1024, H), lambda i: (i, 0)),
            pl.BlockSpec((1024, H), lambda i: (i, 0)),
            pl.BlockSpec((1024, 1), lambda i: (i, 0)),
        ],
        out_shape=[
            jax.ShapeDtypeStruct((P, H), jnp.float32),
            jax.ShapeDtypeStruct((P, H), jnp.float32),
            jax.ShapeDtypeStruct((P, 1), jnp.float32),
        ],
    )(x, Wl, bl.reshape(1, H), Wr, br.reshape(1, H), att.reshape(H, 1))


def _sc_gsum_body(xl_hbm, xr_hbm, src_hbm, dst_hbm, msum_hbm,
                  src_t, dst_t, xlr_v, xrr_v):
    c = lax.axis_index("c")
    s = lax.axis_index("s")
    wid = s * 2 + c
    base = wid * EPT

    pltpu.sync_copy(src_hbm.at[wid], src_t)
    pltpu.sync_copy(dst_hbm.at[wid], dst_t)

    @pl.loop(0, NCH)
    def _(ci):
        off = base + ci * CH
        pltpu.sync_copy(xl_hbm.at[src_t.at[ci]], xlr_v)
        pltpu.sync_copy(xr_hbm.at[dst_t.at[ci]], xrr_v)

        @pl.loop(0, CH)
        def _(e):
            for kk in range(8):
                xlr_v[e, pl.ds(kk * 16, 16)] = (
                    xlr_v[e, pl.ds(kk * 16, 16)] + xrr_v[e, pl.ds(kk * 16, 16)])

        pltpu.sync_copy(xlr_v, msum_hbm.at[pl.ds(off, CH)])


def _sc_gsum(xl, xr, src2d, dst2d):
    kfn = pl.kernel(
        _sc_gsum_body,
        out_type=jax.ShapeDtypeStruct((E, H), jnp.float32),
        mesh=_mesh,
        compiler_params=_SC_PARAMS,
        scratch_types=[
            pltpu.VMEM((NCH, CH), jnp.int32),
            pltpu.VMEM((NCH, CH), jnp.int32),
            pltpu.VMEM((CH, H), jnp.float32),
            pltpu.VMEM((CH, H), jnp.float32),
        ],
    )
    return kfn(xl, xr, src2d, dst2d)


EB = 2000


def _alpha_kernel(ms_ref, ew_ref, we_ref, att_ref, o_ref):
    m = ms_ref[...] + ew_ref[...] * we_ref[...]
    m = jnp.where(m >= 0.0, m, 0.2 * m)
    mb = m.astype(jnp.bfloat16)
    o_ref[...] = jnp.dot(mb, att_ref[...].astype(jnp.bfloat16),
                         preferred_element_type=jnp.float32)


def _alpha_tc(msum, ew, We, att):
    grid = (E // EB,)
    return pl.pallas_call(
        _alpha_kernel,
        grid=grid,
        in_specs=[
            pl.BlockSpec((EB, H), lambda i: (i, 0)),
            pl.BlockSpec((EB, 1), lambda i: (i, 0)),
            pl.BlockSpec((1, H), lambda i: (0, 0)),
            pl.BlockSpec((H, 1), lambda i: (0, 0)),
        ],
        out_specs=pl.BlockSpec((EB, 1), lambda i: (i, 0)),
        out_shape=jax.ShapeDtypeStruct((E, 1), jnp.float32),
    )(msum, ew.reshape(E, 1), We.reshape(1, H), att.reshape(H, 1))


def _sc_segmax_body(dst_hbm, alpha_hbm, amax_part_hbm, dst_t, alpha_t, amax_v):
    c = lax.axis_index("c")
    s = lax.axis_index("s")
    wid = s * 2 + c

    pltpu.sync_copy(dst_hbm.at[wid], dst_t)
    pltpu.sync_copy(alpha_hbm.at[wid], alpha_t)

    neg16 = jnp.full((16,), NEG, jnp.float32)

    @pl.loop(0, PR)
    def _(i):
        @pl.loop(0, 128, step=16)
        def _(j):
            amax_v[i, pl.ds(j, 16)] = neg16

    lane = jnp.arange(16, dtype=jnp.int32)
    rot1 = (lane + 1) & 15

    @pl.loop(0, NCH)
    def _(ci):
        @pl.loop(0, CH // 16)
        def _(g):
            dst16 = dst_t[ci, pl.ds(g * 16, 16)]
            aph = alpha_t[ci, pl.ds(g * 16, 16)]
            kr = dst16
            vr = aph
            vmax = aph
            for _r in range(15):
                kr = _lane_gather(kr, rot1)
                vr = _lane_gather(vr, rot1)
                vmax = jnp.where(kr == dst16, jnp.maximum(vmax, vr), vmax)
            i0 = lax.shift_right_logical(dst16, 7)
            i1 = lax.bitwise_and(dst16, 127)
            cur = plsc.load_gather(amax_v, [i0, i1])
            plsc.store_scatter(amax_v, [i0, i1], jnp.maximum(cur, vmax))

    pltpu.sync_copy(amax_v, amax_part_hbm.at[wid])


def _sc_segmax(dst2d, alpha2d):
    kfn = pl.kernel(
        _sc_segmax_body,
        out_type=jax.ShapeDtypeStruct((NTILES, PR, 128), jnp.float32),
        mesh=_mesh,
        compiler_params=_SC_PARAMS,
        scratch_types=[
            pltpu.VMEM((NCH, CH), jnp.int32),
            pltpu.VMEM((NCH, CH), jnp.float32),
            pltpu.VMEM((PR, 128), jnp.float32),
        ],
    )
    return kfn(dst2d, alpha2d)


def _amax_kernel(ap_ref, sa_ref, o_ref):
    o_ref[...] = jnp.maximum(jnp.max(ap_ref[...], axis=0), sa_ref[...])


def _amax_combine(amax_part, salpha):
    return pl.pallas_call(
        _amax_kernel,
        out_shape=jax.ShapeDtypeStruct((PR, 128), jnp.float32),
    )(amax_part, salpha.reshape(PR, 128))


def _sc_pass_b_body(xl_hbm, src_hbm, dst_hbm, alpha_hbm, amaxf_hbm, zeros_hbm,
                    den_part_hbm, acc_part_hbm,
                    src_v, dst_v, alpha_v, amax_v, den_v, rows_v, acc_sh):
    c = lax.axis_index("c")
    s = lax.axis_index("s")
    wid = s * 2 + c

    pltpu.sync_copy(amaxf_hbm, amax_v)

    z16 = jnp.zeros((16,), jnp.float32)

    @pl.loop(0, PR)
    def _(i):
        @pl.loop(0, 128, step=16)
        def _(j):
            den_v[i, pl.ds(j, 16)] = z16

    pltpu.sync_copy(zeros_hbm, acc_sh.at[pl.ds(s * (P // 16), P // 16)])
    plsc.subcore_barrier()

    lane = jnp.arange(16, dtype=jnp.int32)
    rot1 = (lane + 1) & 15

    @pl.loop(0, NCH)
    def _(ci):
        pltpu.sync_copy(src_hbm.at[wid, ci], src_v)
        pltpu.sync_copy(dst_hbm.at[wid, ci], dst_v)
        pltpu.sync_copy(alpha_hbm.at[wid, ci], alpha_v)
        pltpu.sync_copy(xl_hbm.at[src_v], rows_v)

        @pl.loop(0, CH // 16)
        def _(g):
            dst16 = dst_v[pl.ds(g * 16, 16)]
            a16 = alpha_v[pl.ds(g * 16, 16)]
            i0 = lax.shift_right_logical(dst16, 7)
            i1 = lax.bitwise_and(dst16, 127)
            av = plsc.load_gather(amax_v, [i0, i1])
            exv = jnp.exp(a16 - av)

            kr = dst16
            vr = exv
            vsum = exv
            for _r in range(15):
                kr = _lane_gather(kr, rot1)
                vr = _lane_gather(vr, rot1)
                vsum = jnp.where(kr == dst16, vsum + vr, vsum)
            cur = plsc.load_gather(den_v, [i0, i1])
            plsc.store_scatter(den_v, [i0, i1], cur + vsum)

            for l in range(16):
                e = g * 16 + l
                b = _lane_gather(exv, jnp.full((16,), l, jnp.int32))
                for kk in range(8):
                    rows_v[e, pl.ds(kk * 16, 16)] = (
                        rows_v[e, pl.ds(kk * 16, 16)] * b)

        pltpu.sync_copy(rows_v, acc_sh.at[dst_v], add=True)

    plsc.subcore_barrier()
    pltpu.sync_copy(acc_sh.at[pl.ds(s * (P // 16), P // 16)],
                    acc_part_hbm.at[c, pl.ds(s * (P // 16), P // 16)])
    pltpu.sync_copy(den_v, den_part_hbm.at[wid])


def _sc_pass_b(xl, src2d, dst2d, alpha2d, amaxf, zeros_block):
    kfn = pl.kernel(
        _sc_pass_b_body,
        out_type=[
            jax.ShapeDtypeStruct((NTILES, PR, 128), jnp.float32),
            jax.ShapeDtypeStruct((2, P, H), jnp.float32),
        ],
        mesh=_mesh,
        compiler_params=_SC_PARAMS,
        scratch_types=[
            pltpu.VMEM((CH,), jnp.int32),
            pltpu.VMEM((CH,), jnp.int32),
            pltpu.VMEM((CH,), jnp.float32),
            pltpu.VMEM((PR, 128), jnp.float32),
            pltpu.VMEM((PR, 128), jnp.float32),
            pltpu.VMEM((CH, H), jnp.float32),
            pltpu.VMEM_SHARED((P, H), jnp.float32),
        ],
    )
    return kfn(xl, src2d, dst2d, alpha2d, amaxf, zeros_block)


def _densum_kernel(dp_ref, o_ref):
    o_ref[...] = jnp.sum(dp_ref[...], axis=0)


def _den_combine(den_part):
    return pl.pallas_call(
        _densum_kernel,
        out_shape=jax.ShapeDtypeStruct((PR, 128), jnp.float32),
    )(den_part)


def _finalize_kernel(acc_ref, dr_ref, sa_ref, am_ref, xl_ref, b_ref, o_ref, *,
                     relu):
    exs = jnp.exp(sa_ref[...] - am_ref[...])
    den = dr_ref[...] + exs
    r = 1.0 / (den + 1e-16)
    h = (acc_ref[0] + acc_ref[1] + exs * xl_ref[...]) * r + b_ref[...]
    if relu:
        h = jnp.maximum(h, 0.0)
    o_ref[...] = h


def _finalize(acc_part, den_red, salpha, amaxf, xl, bias, relu):
    grid = (P // 1024,)
    return pl.pallas_call(
        functools.partial(_finalize_kernel, relu=relu),
        grid=grid,
        in_specs=[
            pl.BlockSpec((2, 1024, H), lambda i: (0, i, 0)),
            pl.BlockSpec((1024, 1), lambda i: (i, 0)),
            pl.BlockSpec((1024, 1), lambda i: (i, 0)),
            pl.BlockSpec((1024, 1), lambda i: (i, 0)),
            pl.BlockSpec((1024, H), lambda i: (i, 0)),
            pl.BlockSpec((1, H), lambda i: (0, 0)),
        ],
        out_specs=pl.BlockSpec((1024, H), lambda i: (i, 0)),
        out_shape=jax.ShapeDtypeStruct((P, H), jnp.float32),
    )(acc_part, den_red, salpha, amaxf, xl, bias.reshape(1, H))


def _outproj_kernel(h_ref, w_ref, b_ref, o_ref):
    o_ref[...] = jnp.dot(h_ref[...].astype(jnp.bfloat16),
                         w_ref[...].astype(jnp.bfloat16),
                         preferred_element_type=jnp.float32) + b_ref[0, 0]


def _outproj(h, Wout, bout):
    grid = (P // 1024,)
    return pl.pallas_call(
        _outproj_kernel,
        grid=grid,
        in_specs=[
            pl.BlockSpec((1024, H), lambda i: (i, 0)),
            pl.BlockSpec((H, 1), lambda i: (0, 0)),
            pl.BlockSpec((1, 1), lambda i: (0, 0)),
        ],
        out_specs=pl.BlockSpec((1024, 1), lambda i: (i, 0)),
        out_shape=jax.ShapeDtypeStruct((P, 1), jnp.float32),
    )(h, Wout.reshape(H, 1), bout.reshape(1, 1))


def _gat_layer_sc(x, src, dst, ew, zeros_block, Wl, bl, Wr, br, We, att, bias,
                  relu):
    xl, xr, salpha = _proj(x, Wl, bl, Wr, br, att)
    src2d = src.reshape(NTILES, NCH, CH)
    dst2d = dst.reshape(NTILES, NCH, CH)
    msum = _sc_gsum(xl, xr, src2d, dst2d)
    alpha2d = _alpha_tc(msum, ew, We, att).reshape(NTILES, NCH, CH)
    amax_part = _sc_segmax(dst2d, alpha2d)
    amaxf = _amax_combine(amax_part, salpha)
    den_part, acc_part = _sc_pass_b(xl, src2d, dst2d, alpha2d, amaxf,
                                    zeros_block)
    den_red = _den_combine(den_part)
    return _finalize(acc_part, den_red.reshape(P, 1), salpha,
                     amaxf.reshape(P, 1), xl, bias, relu)


def kernel(x, edge_index, edge_attr, Wl1, bl1, Wr1, br1, We1, att1, b1, Wl2,
           bl2, Wr2, br2, We2, att2, b2, Wl3, bl3, Wr3, br3, We3, att3, b3,
           Wout, bout):
    src = edge_index[0]
    dst = edge_index[1]
    ew = _ew_transform(edge_attr.reshape(E))
    x_pad = jnp.zeros((P, x.shape[1]), jnp.float32).at[:N].set(x)
    zeros_block = jnp.zeros((P // 16, H), jnp.float32)

    h = _gat_layer_sc(x_pad, src, dst, ew, zeros_block, Wl1, bl1, Wr1, br1,
                      We1.reshape(H), att1, b1, relu=True)
    h = _gat_layer_sc(h, src, dst, ew, zeros_block, Wl2, bl2, Wr2, br2,
                      We2.reshape(H), att2, b2, relu=True)
    h = _gat_layer_sc(h, src, dst, ew, zeros_block, Wl3, bl3, Wr3, br3,
                      We3.reshape(H), att3, b3, relu=False)
    out = _outproj(h, Wout, bout)
    return out.reshape(P)[:N]


# speedup vs baseline: 7.2097x; 1.2020x over previous
import dataclasses
import functools

import jax
import jax.numpy as jnp
from jax import lax
from jax.experimental import pallas as pl
from jax.experimental.pallas import tpu as pltpu
from jax.experimental.pallas import tpu_sc as plsc

N = 10000
E = 320000
H = 128
P = 10240
PR = P // 128
NTILES = 32
EPT = E // NTILES
CH = 80
NCH = EPT // CH
NEG = -1.0e30

_mesh = plsc.VectorSubcoreMesh(core_axis_name="c", subcore_axis_name="s")

_SC_PARAMS = pltpu.CompilerParams()
if "needs_layout_passes" in pltpu.CompilerParams.__dataclass_fields__:
    _SC_PARAMS = dataclasses.replace(_SC_PARAMS, needs_layout_passes=False)

_GATHER_DNUMS = lax.GatherDimensionNumbers(
    offset_dims=(), collapsed_slice_dims=(0,), start_index_map=(0,))


def _lane_gather(v, idx):
    return lax.gather(v, idx[:, None], _GATHER_DNUMS, slice_sizes=(1,),
                      mode=lax.GatherScatterMode.PROMISE_IN_BOUNDS)


def _bf16_round(v):
    u = plsc.bitcast(v, jnp.uint32)
    r = lax.bitwise_and(lax.shift_right_logical(u, jnp.uint32(16)),
                        jnp.uint32(1))
    u = u + jnp.uint32(0x7FFF) + r
    u = lax.bitwise_and(u, jnp.uint32(0xFFFF0000))
    return plsc.bitcast(u, jnp.float32)


def _ew_kernel(ea_ref, o_ref):
    ea = ea_ref[...]
    safe = jnp.where(ea == 0.0, 1.0, ea)
    o_ref[...] = jnp.where(ea == 0.0, 1.0 / 0.0001, 1.0 / safe)


def _ew_transform(edge_attr):
    ea = edge_attr.reshape(E // 128, 128)
    out = pl.pallas_call(
        _ew_kernel,
        out_shape=jax.ShapeDtypeStruct((E // 128, 128), jnp.float32),
    )(ea)
    return out.reshape(E)


def _proj_kernel(x_ref, wl_ref, bl_ref, wr_ref, br_ref, att_ref, xl_ref,
                 xr_ref, sa_ref):
    xb = x_ref[...].astype(jnp.bfloat16)
    xl = jnp.dot(xb, wl_ref[...].astype(jnp.bfloat16),
                 preferred_element_type=jnp.float32) + bl_ref[...]
    xr = jnp.dot(xb, wr_ref[...].astype(jnp.bfloat16),
                 preferred_element_type=jnp.float32) + br_ref[...]
    xl_ref[...] = xl
    xr_ref[...] = xr
    m = xl + xr
    m = jnp.where(m >= 0.0, m, 0.2 * m)
    sa_ref[...] = jnp.dot(m.astype(jnp.bfloat16),
                          att_ref[...].astype(jnp.bfloat16),
                          preferred_element_type=jnp.float32)


def _proj(x, Wl, bl, Wr, br, att):
    grid = (P // 1024,)
    return pl.pallas_call(
        _proj_kernel,
        grid=grid,
        in_specs=[
            pl.BlockSpec((1024, H), lambda i: (i, 0)),
            pl.BlockSpec((H, H), lambda i: (0, 0)),
            pl.BlockSpec((1, H), lambda i: (0, 0)),
            pl.BlockSpec((H, H), lambda i: (0, 0)),
            pl.BlockSpec((1, H), lambda i: (0, 0)),
            pl.BlockSpec((H, 1), lambda i: (0, 0)),
        ],
        out_specs=[
            pl.BlockSpec((1024, H), lambda i: (i, 0)),
            pl.BlockSpec((1024, H), lambda i: (i, 0)),
            pl.BlockSpec((1024, 1), lambda i: (i, 0)),
        ],
        out_shape=[
            jax.ShapeDtypeStruct((P, H), jnp.float32),
            jax.ShapeDtypeStruct((P, H), jnp.float32),
            jax.ShapeDtypeStruct((P, 1), jnp.float32),
        ],
    )(x, Wl, bl.reshape(1, H), Wr, br.reshape(1, H), att.reshape(H, 1))


def _sc_gsum_body(xl_hbm, xr_hbm, src_hbm, dst_hbm, msum_hbm,
                  src_t, dst_t, xlr_v, xrr_v, sem1, sem2):
    c = lax.axis_index("c")
    s = lax.axis_index("s")
    wid = s * 2 + c
    base = wid * EPT

    pltpu.sync_copy(src_hbm.at[wid], src_t)
    pltpu.sync_copy(dst_hbm.at[wid], dst_t)

    @pl.loop(0, NCH)
    def _(ci):
        off = base + ci * CH
        cpa = pltpu.async_copy(xl_hbm.at[src_t.at[ci]], xlr_v, sem1)
        cpb = pltpu.async_copy(xr_hbm.at[dst_t.at[ci]], xrr_v, sem2)
        cpa.wait()
        cpb.wait()

        @pl.loop(0, CH)
        def _(e):
            for kk in range(8):
                xlr_v[e, pl.ds(kk * 16, 16)] = (
                    xlr_v[e, pl.ds(kk * 16, 16)] + xrr_v[e, pl.ds(kk * 16, 16)])

        pltpu.sync_copy(xlr_v, msum_hbm.at[pl.ds(off, CH)])


def _sc_gsum(xl, xr, src2d, dst2d):
    kfn = pl.kernel(
        _sc_gsum_body,
        out_type=jax.ShapeDtypeStruct((E, H), jnp.float32),
        mesh=_mesh,
        compiler_params=_SC_PARAMS,
        scratch_types=[
            pltpu.VMEM((NCH, CH), jnp.int32),
            pltpu.VMEM((NCH, CH), jnp.int32),
            pltpu.VMEM((CH, H), jnp.float32),
            pltpu.VMEM((CH, H), jnp.float32),
            pltpu.SemaphoreType.DMA,
            pltpu.SemaphoreType.DMA,
        ],
    )
    return kfn(xl, xr, src2d, dst2d)


EB = 2000


def _alpha_kernel(ms_ref, ew_ref, we_ref, att_ref, o_ref):
    m = ms_ref[...] + ew_ref[...] * we_ref[...]
    m = jnp.where(m >= 0.0, m, 0.2 * m)
    mb = m.astype(jnp.bfloat16)
    o_ref[...] = jnp.dot(mb, att_ref[...].astype(jnp.bfloat16),
                         preferred_element_type=jnp.float32)


def _alpha_tc(msum, ew, We, att):
    grid = (E // EB,)
    return pl.pallas_call(
        _alpha_kernel,
        grid=grid,
        in_specs=[
            pl.BlockSpec((EB, H), lambda i: (i, 0)),
            pl.BlockSpec((EB, 1), lambda i: (i, 0)),
            pl.BlockSpec((1, H), lambda i: (0, 0)),
            pl.BlockSpec((H, 1), lambda i: (0, 0)),
        ],
        out_specs=pl.BlockSpec((EB, 1), lambda i: (i, 0)),
        out_shape=jax.ShapeDtypeStruct((E, 1), jnp.float32),
    )(msum, ew.reshape(E, 1), We.reshape(1, H), att.reshape(H, 1))


def _sc_segmax_body(dst_hbm, alpha_hbm, amax_part_hbm, dst_t, alpha_t, amax_v):
    c = lax.axis_index("c")
    s = lax.axis_index("s")
    wid = s * 2 + c

    pltpu.sync_copy(dst_hbm.at[wid], dst_t)
    pltpu.sync_copy(alpha_hbm.at[wid], alpha_t)

    neg16 = jnp.full((16,), NEG, jnp.float32)

    @pl.loop(0, PR)
    def _(i):
        @pl.loop(0, 128, step=16)
        def _(j):
            amax_v[i, pl.ds(j, 16)] = neg16

    lane = jnp.arange(16, dtype=jnp.int32)
    rot1 = (lane + 1) & 15

    @pl.loop(0, NCH)
    def _(ci):
        @pl.loop(0, CH // 16)
        def _(g):
            dst16 = dst_t[ci, pl.ds(g * 16, 16)]
            aph = alpha_t[ci, pl.ds(g * 16, 16)]
            kr = dst16
            vr = aph
            vmax = aph
            for _r in range(15):
                kr = _lane_gather(kr, rot1)
                vr = _lane_gather(vr, rot1)
                vmax = jnp.where(kr == dst16, jnp.maximum(vmax, vr), vmax)
            i0 = lax.shift_right_logical(dst16, 7)
            i1 = lax.bitwise_and(dst16, 127)
            cur = plsc.load_gather(amax_v, [i0, i1])
            plsc.store_scatter(amax_v, [i0, i1], jnp.maximum(cur, vmax))

    pltpu.sync_copy(amax_v, amax_part_hbm.at[wid])


def _sc_segmax(dst2d, alpha2d):
    kfn = pl.kernel(
        _sc_segmax_body,
        out_type=jax.ShapeDtypeStruct((NTILES, PR, 128), jnp.float32),
        mesh=_mesh,
        compiler_params=_SC_PARAMS,
        scratch_types=[
            pltpu.VMEM((NCH, CH), jnp.int32),
            pltpu.VMEM((NCH, CH), jnp.float32),
            pltpu.VMEM((PR, 128), jnp.float32),
        ],
    )
    return kfn(dst2d, alpha2d)


def _amax_kernel(ap_ref, sa_ref, o_ref):
    o_ref[...] = jnp.maximum(jnp.max(ap_ref[...], axis=0), sa_ref[...])


def _amax_combine(amax_part, salpha):
    return pl.pallas_call(
        _amax_kernel,
        out_shape=jax.ShapeDtypeStruct((PR, 128), jnp.float32),
    )(amax_part, salpha.reshape(PR, 128))


def _sc_pass_b_body(xl_hbm, src_hbm, dst_hbm, alpha_hbm, amaxf_hbm, zeros_hbm,
                    den_part_hbm, acc_part_hbm,
                    src_v, dst_v, alpha_v, amax_v, den_v, rows_v, acc_sh,
                    sem1, sem2, sem3):
    c = lax.axis_index("c")
    s = lax.axis_index("s")
    wid = s * 2 + c

    pltpu.sync_copy(amaxf_hbm, amax_v)

    z16 = jnp.zeros((16,), jnp.float32)

    @pl.loop(0, PR)
    def _(i):
        @pl.loop(0, 128, step=16)
        def _(j):
            den_v[i, pl.ds(j, 16)] = z16

    pltpu.sync_copy(zeros_hbm, acc_sh.at[pl.ds(s * (P // 16), P // 16)])
    plsc.subcore_barrier()

    lane = jnp.arange(16, dtype=jnp.int32)
    rot1 = (lane + 1) & 15

    @pl.loop(0, NCH)
    def _(ci):
        cpa = pltpu.async_copy(src_hbm.at[wid, ci], src_v, sem1)
        cpb = pltpu.async_copy(dst_hbm.at[wid, ci], dst_v, sem2)
        cpc = pltpu.async_copy(alpha_hbm.at[wid, ci], alpha_v, sem3)
        cpa.wait()
        cpb.wait()
        cpc.wait()
        pltpu.sync_copy(xl_hbm.at[src_v], rows_v)

        @pl.loop(0, CH // 16)
        def _(g):
            dst16 = dst_v[pl.ds(g * 16, 16)]
            a16 = alpha_v[pl.ds(g * 16, 16)]
            i0 = lax.shift_right_logical(dst16, 7)
            i1 = lax.bitwise_and(dst16, 127)
            av = plsc.load_gather(amax_v, [i0, i1])
            exv = jnp.exp(a16 - av)

            kr = dst16
            vr = exv
            vsum = exv
            for _r in range(15):
                kr = _lane_gather(kr, rot1)
                vr = _lane_gather(vr, rot1)
                vsum = jnp.where(kr == dst16, vsum + vr, vsum)
            cur = plsc.load_gather(den_v, [i0, i1])
            plsc.store_scatter(den_v, [i0, i1], cur + vsum)

            for l in range(16):
                e = g * 16 + l
                b = _lane_gather(exv, jnp.full((16,), l, jnp.int32))
                for kk in range(8):
                    rows_v[e, pl.ds(kk * 16, 16)] = (
                        rows_v[e, pl.ds(kk * 16, 16)] * b)

        pltpu.sync_copy(rows_v, acc_sh.at[dst_v], add=True)

    plsc.subcore_barrier()
    pltpu.sync_copy(acc_sh.at[pl.ds(s * (P // 16), P // 16)],
                    acc_part_hbm.at[c, pl.ds(s * (P // 16), P // 16)])
    pltpu.sync_copy(den_v, den_part_hbm.at[wid])


def _sc_pass_b(xl, src2d, dst2d, alpha2d, amaxf, zeros_block):
    kfn = pl.kernel(
        _sc_pass_b_body,
        out_type=[
            jax.ShapeDtypeStruct((NTILES, PR, 128), jnp.float32),
            jax.ShapeDtypeStruct((2, P, H), jnp.float32),
        ],
        mesh=_mesh,
        compiler_params=_SC_PARAMS,
        scratch_types=[
            pltpu.VMEM((CH,), jnp.int32),
            pltpu.VMEM((CH,), jnp.int32),
            pltpu.VMEM((CH,), jnp.float32),
            pltpu.VMEM((PR, 128), jnp.float32),
            pltpu.VMEM((PR, 128), jnp.float32),
            pltpu.VMEM((CH, H), jnp.float32),
            pltpu.VMEM_SHARED((P, H), jnp.float32),
            pltpu.SemaphoreType.DMA,
            pltpu.SemaphoreType.DMA,
            pltpu.SemaphoreType.DMA,
        ],
    )
    return kfn(xl, src2d, dst2d, alpha2d, amaxf, zeros_block)


def _densum_kernel(dp_ref, o_ref):
    o_ref[...] = jnp.sum(dp_ref[...], axis=0)


def _den_combine(den_part):
    return pl.pallas_call(
        _densum_kernel,
        out_shape=jax.ShapeDtypeStruct((PR, 128), jnp.float32),
    )(den_part)


def _finalize_kernel(acc_ref, dr_ref, sa_ref, am_ref, xl_ref, b_ref, o_ref, *,
                     relu):
    exs = jnp.exp(sa_ref[...] - am_ref[...])
    den = dr_ref[...] + exs
    r = 1.0 / (den + 1e-16)
    h = (acc_ref[0] + acc_ref[1] + exs * xl_ref[...]) * r + b_ref[...]
    if relu:
        h = jnp.maximum(h, 0.0)
    o_ref[...] = h


def _finalize(acc_part, den_red, salpha, amaxf, xl, bias, relu):
    grid = (P // 1024,)
    return pl.pallas_call(
        functools.partial(_finalize_kernel, relu=relu),
        grid=grid,
        in_specs=[
            pl.BlockSpec((2, 1024, H), lambda i: (0, i, 0)),
            pl.BlockSpec((1024, 1), lambda i: (i, 0)),
            pl.BlockSpec((1024, 1), lambda i: (i, 0)),
            pl.BlockSpec((1024, 1), lambda i: (i, 0)),
            pl.BlockSpec((1024, H), lambda i: (i, 0)),
            pl.BlockSpec((1, H), lambda i: (0, 0)),
        ],
        out_specs=pl.BlockSpec((1024, H), lambda i: (i, 0)),
        out_shape=jax.ShapeDtypeStruct((P, H), jnp.float32),
    )(acc_part, den_red, salpha, amaxf, xl, bias.reshape(1, H))


def _outproj_kernel(h_ref, w_ref, b_ref, o_ref):
    o_ref[...] = jnp.dot(h_ref[...].astype(jnp.bfloat16),
                         w_ref[...].astype(jnp.bfloat16),
                         preferred_element_type=jnp.float32) + b_ref[0, 0]


def _outproj(h, Wout, bout):
    grid = (P // 1024,)
    return pl.pallas_call(
        _outproj_kernel,
        grid=grid,
        in_specs=[
            pl.BlockSpec((1024, H), lambda i: (i, 0)),
            pl.BlockSpec((H, 1), lambda i: (0, 0)),
            pl.BlockSpec((1, 1), lambda i: (0, 0)),
        ],
        out_specs=pl.BlockSpec((1024, 1), lambda i: (i, 0)),
        out_shape=jax.ShapeDtypeStruct((P, 1), jnp.float32),
    )(h, Wout.reshape(H, 1), bout.reshape(1, 1))


def _gat_layer_sc(x, src, dst, ew, zeros_block, Wl, bl, Wr, br, We, att, bias,
                  relu):
    xl, xr, salpha = _proj(x, Wl, bl, Wr, br, att)
    src2d = src.reshape(NTILES, NCH, CH)
    dst2d = dst.reshape(NTILES, NCH, CH)
    msum = _sc_gsum(xl, xr, src2d, dst2d)
    alpha2d = _alpha_tc(msum, ew, We, att).reshape(NTILES, NCH, CH)
    amax_part = _sc_segmax(dst2d, alpha2d)
    amaxf = _amax_combine(amax_part, salpha)
    den_part, acc_part = _sc_pass_b(xl, src2d, dst2d, alpha2d, amaxf,
                                    zeros_block)
    den_red = _den_combine(den_part)
    return _finalize(acc_part, den_red.reshape(P, 1), salpha,
                     amaxf.reshape(P, 1), xl, bias, relu)


def kernel(x, edge_index, edge_attr, Wl1, bl1, Wr1, br1, We1, att1, b1, Wl2,
           bl2, Wr2, br2, We2, att2, b2, Wl3, bl3, Wr3, br3, We3, att3, b3,
           Wout, bout):
    src = edge_index[0]
    dst = edge_index[1]
    ew = _ew_transform(edge_attr.reshape(E))
    x_pad = jnp.zeros((P, x.shape[1]), jnp.float32).at[:N].set(x)
    zeros_block = jnp.zeros((P // 16, H), jnp.float32)

    h = _gat_layer_sc(x_pad, src, dst, ew, zeros_block, Wl1, bl1, Wr1, br1,
                      We1.reshape(H), att1, b1, relu=True)
    h = _gat_layer_sc(h, src, dst, ew, zeros_block, Wl2, bl2, Wr2, br2,
                      We2.reshape(H), att2, b2, relu=True)
    h = _gat_layer_sc(h, src, dst, ew, zeros_block, Wl3, bl3, Wr3, br3,
                      We3.reshape(H), att3, b3, relu=False)
    out = _outproj(h, Wout, bout)
    return out.reshape(P)[:N]


# speedup vs baseline: 7.2340x; 1.0034x over previous
import dataclasses
import functools

import jax
import jax.numpy as jnp
from jax import lax
from jax.experimental import pallas as pl
from jax.experimental.pallas import tpu as pltpu
from jax.experimental.pallas import tpu_sc as plsc

N = 10000
E = 320000
H = 128
P = 10240
PR = P // 128
NTILES = 32
EPT = E // NTILES
CH = 80
NCH = EPT // CH
NEG = -1.0e30

_mesh = plsc.VectorSubcoreMesh(core_axis_name="c", subcore_axis_name="s")

_SC_PARAMS = pltpu.CompilerParams()
if "needs_layout_passes" in pltpu.CompilerParams.__dataclass_fields__:
    _SC_PARAMS = dataclasses.replace(_SC_PARAMS, needs_layout_passes=False)

_GATHER_DNUMS = lax.GatherDimensionNumbers(
    offset_dims=(), collapsed_slice_dims=(0,), start_index_map=(0,))


def _lane_gather(v, idx):
    return lax.gather(v, idx[:, None], _GATHER_DNUMS, slice_sizes=(1,),
                      mode=lax.GatherScatterMode.PROMISE_IN_BOUNDS)


def _ew_kernel(ea_ref, o_ref):
    ea = ea_ref[...]
    safe = jnp.where(ea == 0.0, 1.0, ea)
    o_ref[...] = jnp.where(ea == 0.0, 1.0 / 0.0001, 1.0 / safe)


def _ew_transform(edge_attr):
    ea = edge_attr.reshape(E // 128, 128)
    out = pl.pallas_call(
        _ew_kernel,
        out_shape=jax.ShapeDtypeStruct((E // 128, 128), jnp.float32),
    )(ea)
    return out.reshape(E)


def _proj_kernel(x_ref, wl_ref, bl_ref, wr_ref, br_ref, att_ref, xl_ref,
                 xr_ref, sa_ref):
    xb = x_ref[...].astype(jnp.bfloat16)
    xl = jnp.dot(xb, wl_ref[...].astype(jnp.bfloat16),
                 preferred_element_type=jnp.float32) + bl_ref[...]
    xr = jnp.dot(xb, wr_ref[...].astype(jnp.bfloat16),
                 preferred_element_type=jnp.float32) + br_ref[...]
    xl_ref[...] = xl
    xr_ref[...] = xr
    m = xl + xr
    m = jnp.where(m >= 0.0, m, 0.2 * m)
    sa_ref[...] = jnp.dot(m.astype(jnp.bfloat16),
                          att_ref[...].astype(jnp.bfloat16),
                          preferred_element_type=jnp.float32)


def _proj(x, Wl, bl, Wr, br, att):
    grid = (P // 1024,)
    return pl.pallas_call(
        _proj_kernel,
        grid=grid,
        in_specs=[
            pl.BlockSpec((1024, H), lambda i: (i, 0)),
            pl.BlockSpec((H, H), lambda i: (0, 0)),
            pl.BlockSpec((1, H), lambda i: (0, 0)),
            pl.BlockSpec((H, H), lambda i: (0, 0)),
            pl.BlockSpec((1, H), lambda i: (0, 0)),
            pl.BlockSpec((H, 1), lambda i: (0, 0)),
        ],
        out_specs=[
            pl.BlockSpec((1024, H), lambda i: (i, 0)),
            pl.BlockSpec((1024, H), lambda i: (i, 0)),
            pl.BlockSpec((1024, 1), lambda i: (i, 0)),
        ],
        out_shape=[
            jax.ShapeDtypeStruct((P, H), jnp.float32),
            jax.ShapeDtypeStruct((P, H), jnp.float32),
            jax.ShapeDtypeStruct((P, 1), jnp.float32),
        ],
    )(x, Wl, bl.reshape(1, H), Wr, br.reshape(1, H), att.reshape(H, 1))


def _sc_gsum_body(xl_hbm, xr_hbm, src_hbm, dst_hbm, msum_hbm,
                  src_t, dst_t, xlr_v, xrr_v, sem1, sem2):
    c = lax.axis_index("c")
    s = lax.axis_index("s")
    wid = s * 2 + c
    base = wid * EPT

    pltpu.sync_copy(src_hbm.at[wid], src_t)
    pltpu.sync_copy(dst_hbm.at[wid], dst_t)

    @pl.loop(0, NCH)
    def _(ci):
        off = base + ci * CH
        cpa = pltpu.async_copy(xl_hbm.at[src_t.at[ci]], xlr_v, sem1)
        cpb = pltpu.async_copy(xr_hbm.at[dst_t.at[ci]], xrr_v, sem2)
        cpa.wait()
        cpb.wait()

        @pl.loop(0, CH)
        def _(e):
            for kk in range(8):
                xlr_v[e, pl.ds(kk * 16, 16)] = (
                    xlr_v[e, pl.ds(kk * 16, 16)] + xrr_v[e, pl.ds(kk * 16, 16)])

        pltpu.sync_copy(xlr_v, msum_hbm.at[pl.ds(off, CH)])


def _sc_gsum(xl, xr, src2d, dst2d):
    kfn = pl.kernel(
        _sc_gsum_body,
        out_type=jax.ShapeDtypeStruct((E, H), jnp.float32),
        mesh=_mesh,
        compiler_params=_SC_PARAMS,
        scratch_types=[
            pltpu.VMEM((NCH, CH), jnp.int32),
            pltpu.VMEM((NCH, CH), jnp.int32),
            pltpu.VMEM((CH, H), jnp.float32),
            pltpu.VMEM((CH, H), jnp.float32),
            pltpu.SemaphoreType.DMA,
            pltpu.SemaphoreType.DMA,
        ],
    )
    return kfn(xl, xr, src2d, dst2d)


EB = 2000


def _alpha_kernel(ms_ref, ew_ref, we_ref, att_ref, o_ref):
    m = ms_ref[...] + ew_ref[...] * we_ref[...]
    m = jnp.where(m >= 0.0, m, 0.2 * m)
    mb = m.astype(jnp.bfloat16)
    o_ref[...] = jnp.dot(mb, att_ref[...].astype(jnp.bfloat16),
                         preferred_element_type=jnp.float32)


def _alpha_tc(msum, ew, We, att):
    grid = (E // EB,)
    return pl.pallas_call(
        _alpha_kernel,
        grid=grid,
        in_specs=[
            pl.BlockSpec((EB, H), lambda i: (i, 0)),
            pl.BlockSpec((EB, 1), lambda i: (i, 0)),
            pl.BlockSpec((1, H), lambda i: (0, 0)),
            pl.BlockSpec((H, 1), lambda i: (0, 0)),
        ],
        out_specs=pl.BlockSpec((EB, 1), lambda i: (i, 0)),
        out_shape=jax.ShapeDtypeStruct((E, 1), jnp.float32),
    )(msum, ew.reshape(E, 1), We.reshape(1, H), att.reshape(H, 1))


def _sc_segmax_body(dst_hbm, alpha_hbm, amax_part_hbm, dst_t, alpha_t, amax_v):
    c = lax.axis_index("c")
    s = lax.axis_index("s")
    wid = s * 2 + c

    pltpu.sync_copy(dst_hbm.at[wid], dst_t)
    pltpu.sync_copy(alpha_hbm.at[wid], alpha_t)

    neg16 = jnp.full((16,), NEG, jnp.float32)

    @pl.loop(0, PR)
    def _(i):
        @pl.loop(0, 128, step=16)
        def _(j):
            amax_v[i, pl.ds(j, 16)] = neg16

    lane = jnp.arange(16, dtype=jnp.int32)
    rot1 = (lane + 1) & 15

    @pl.loop(0, NCH)
    def _(ci):
        @pl.loop(0, CH // 16)
        def _(g):
            dst16 = dst_t[ci, pl.ds(g * 16, 16)]
            aph = alpha_t[ci, pl.ds(g * 16, 16)]
            kr = dst16
            vr = aph
            vmax = aph
            for _r in range(15):
                kr = _lane_gather(kr, rot1)
                vr = _lane_gather(vr, rot1)
                vmax = jnp.where(kr == dst16, jnp.maximum(vmax, vr), vmax)
            i0 = lax.shift_right_logical(dst16, 7)
            i1 = lax.bitwise_and(dst16, 127)
            cur = plsc.load_gather(amax_v, [i0, i1])
            plsc.store_scatter(amax_v, [i0, i1], jnp.maximum(cur, vmax))

    pltpu.sync_copy(amax_v, amax_part_hbm.at[wid])


def _sc_segmax(dst2d, alpha2d):
    kfn = pl.kernel(
        _sc_segmax_body,
        out_type=jax.ShapeDtypeStruct((NTILES, PR, 128), jnp.float32),
        mesh=_mesh,
        compiler_params=_SC_PARAMS,
        scratch_types=[
            pltpu.VMEM((NCH, CH), jnp.int32),
            pltpu.VMEM((NCH, CH), jnp.float32),
            pltpu.VMEM((PR, 128), jnp.float32),
        ],
    )
    return kfn(dst2d, alpha2d)


def _amax_kernel(ap_ref, sa_ref, o_ref):
    o_ref[...] = jnp.maximum(jnp.max(ap_ref[...], axis=0), sa_ref[...])


def _amax_combine(amax_part, salpha):
    return pl.pallas_call(
        _amax_kernel,
        out_shape=jax.ShapeDtypeStruct((PR, 128), jnp.float32),
    )(amax_part, salpha.reshape(PR, 128))


def _sc_pass_b_body(xl_hbm, src_hbm, dst_hbm, alpha_hbm, amaxf_hbm, zeros_hbm,
                    den_part_hbm, acc_part_hbm,
                    src_v, dst_v, alpha_v, amax_v, den_v, rows_v, acc_sh,
                    sem1, sem2, sem3):
    c = lax.axis_index("c")
    s = lax.axis_index("s")
    wid = s * 2 + c

    pltpu.sync_copy(amaxf_hbm, amax_v)

    z16 = jnp.zeros((16,), jnp.float32)

    @pl.loop(0, PR)
    def _(i):
        @pl.loop(0, 128, step=16)
        def _(j):
            den_v[i, pl.ds(j, 16)] = z16

    pltpu.sync_copy(zeros_hbm, acc_sh.at[pl.ds(s * (P // 16), P // 16)])
    plsc.subcore_barrier()

    lane = jnp.arange(16, dtype=jnp.int32)
    rot1 = (lane + 1) & 15

    @pl.loop(0, NCH)
    def _(ci):
        cpa = pltpu.async_copy(src_hbm.at[wid, ci], src_v, sem1)
        cpb = pltpu.async_copy(dst_hbm.at[wid, ci], dst_v, sem2)
        cpc = pltpu.async_copy(alpha_hbm.at[wid, ci], alpha_v, sem3)
        cpa.wait()
        cpb.wait()
        cpc.wait()
        pltpu.sync_copy(xl_hbm.at[src_v], rows_v)

        @pl.loop(0, CH // 16)
        def _(g):
            dst16 = dst_v[pl.ds(g * 16, 16)]
            a16 = alpha_v[pl.ds(g * 16, 16)]
            i0 = lax.shift_right_logical(dst16, 7)
            i1 = lax.bitwise_and(dst16, 127)
            av = plsc.load_gather(amax_v, [i0, i1])
            exv = jnp.exp(a16 - av)

            kr = dst16
            vr = exv
            vsum = exv
            for _r in range(15):
                kr = _lane_gather(kr, rot1)
                vr = _lane_gather(vr, rot1)
                vsum = jnp.where(kr == dst16, vsum + vr, vsum)
            cur = plsc.load_gather(den_v, [i0, i1])
            plsc.store_scatter(den_v, [i0, i1], cur + vsum)

            for l in range(16):
                e = g * 16 + l
                b = _lane_gather(exv, jnp.full((16,), l, jnp.int32))
                for kk in range(8):
                    rows_v[e, pl.ds(kk * 16, 16)] = (
                        rows_v[e, pl.ds(kk * 16, 16)] * b)

        pltpu.sync_copy(rows_v, acc_sh.at[dst_v], add=True)

    plsc.subcore_barrier()
    pltpu.sync_copy(acc_sh.at[pl.ds(s * (P // 16), P // 16)],
                    acc_part_hbm.at[c, pl.ds(s * (P // 16), P // 16)])
    pltpu.sync_copy(den_v, den_part_hbm.at[wid])


def _sc_pass_b(xl, src2d, dst2d, alpha2d, amaxf, zeros_block):
    kfn = pl.kernel(
        _sc_pass_b_body,
        out_type=[
            jax.ShapeDtypeStruct((NTILES, PR, 128), jnp.float32),
            jax.ShapeDtypeStruct((2, P, H), jnp.float32),
        ],
        mesh=_mesh,
        compiler_params=_SC_PARAMS,
        scratch_types=[
            pltpu.VMEM((CH,), jnp.int32),
            pltpu.VMEM((CH,), jnp.int32),
            pltpu.VMEM((CH,), jnp.float32),
            pltpu.VMEM((PR, 128), jnp.float32),
            pltpu.VMEM((PR, 128), jnp.float32),
            pltpu.VMEM((CH, H), jnp.float32),
            pltpu.VMEM_SHARED((P, H), jnp.float32),
            pltpu.SemaphoreType.DMA,
            pltpu.SemaphoreType.DMA,
            pltpu.SemaphoreType.DMA,
        ],
    )
    return kfn(xl, src2d, dst2d, alpha2d, amaxf, zeros_block)


def _densum_kernel(dp_ref, o_ref):
    o_ref[...] = jnp.sum(dp_ref[...], axis=0)


def _den_combine(den_part):
    return pl.pallas_call(
        _densum_kernel,
        out_shape=jax.ShapeDtypeStruct((PR, 128), jnp.float32),
    )(den_part)


def _finalize_kernel(acc_ref, dr_ref, sa_ref, am_ref, xl_ref, b_ref, o_ref, *,
                     relu):
    exs = jnp.exp(sa_ref[...] - am_ref[...])
    den = dr_ref[...] + exs
    r = 1.0 / (den + 1e-16)
    h = (acc_ref[0] + acc_ref[1] + exs * xl_ref[...]) * r + b_ref[...]
    if relu:
        h = jnp.maximum(h, 0.0)
    o_ref[...] = h


def _finalize(acc_part, den_red, salpha, amaxf, xl, bias, relu):
    grid = (P // 1024,)
    return pl.pallas_call(
        functools.partial(_finalize_kernel, relu=relu),
        grid=grid,
        in_specs=[
            pl.BlockSpec((2, 1024, H), lambda i: (0, i, 0)),
            pl.BlockSpec((1024, 1), lambda i: (i, 0)),
            pl.BlockSpec((1024, 1), lambda i: (i, 0)),
            pl.BlockSpec((1024, 1), lambda i: (i, 0)),
            pl.BlockSpec((1024, H), lambda i: (i, 0)),
            pl.BlockSpec((1, H), lambda i: (0, 0)),
        ],
        out_specs=pl.BlockSpec((1024, H), lambda i: (i, 0)),
        out_shape=jax.ShapeDtypeStruct((P, H), jnp.float32),
    )(acc_part, den_red, salpha, amaxf, xl, bias.reshape(1, H))


def _outproj_kernel(h_ref, w_ref, b_ref, o_ref):
    o_ref[...] = jnp.dot(h_ref[...].astype(jnp.bfloat16),
                         w_ref[...].astype(jnp.bfloat16),
                         preferred_element_type=jnp.float32) + b_ref[0, 0]


def _outproj(h, Wout, bout):
    grid = (P // 1024,)
    return pl.pallas_call(
        _outproj_kernel,
        grid=grid,
        in_specs=[
            pl.BlockSpec((1024, H), lambda i: (i, 0)),
            pl.BlockSpec((H, 1), lambda i: (0, 0)),
            pl.BlockSpec((1, 1), lambda i: (0, 0)),
        ],
        out_specs=pl.BlockSpec((1024, 1), lambda i: (i, 0)),
        out_shape=jax.ShapeDtypeStruct((P, 1), jnp.float32),
    )(h, Wout.reshape(H, 1), bout.reshape(1, 1))


def _gat_layer_sc(x, src, dst, ew, zeros_block, Wl, bl, Wr, br, We, att, bias,
                  relu):
    xl, xr, salpha = _proj(x, Wl, bl, Wr, br, att)
    src2d = src.reshape(NTILES, NCH, CH)
    dst2d = dst.reshape(NTILES, NCH, CH)
    msum = _sc_gsum(xl, xr, src2d, dst2d)
    alpha2d = _alpha_tc(msum, ew, We, att).reshape(NTILES, NCH, CH)
    amax_part = _sc_segmax(dst2d, alpha2d)
    amaxf = _amax_combine(amax_part, salpha)
    den_part, acc_part = _sc_pass_b(xl, src2d, dst2d, alpha2d, amaxf,
                                    zeros_block)
    den_red = _den_combine(den_part)
    return _finalize(acc_part, den_red.reshape(P, 1), salpha,
                     amaxf.reshape(P, 1), xl, bias, relu)


def kernel(x, edge_index, edge_attr, Wl1, bl1, Wr1, br1, We1, att1, b1, Wl2,
           bl2, Wr2, br2, We2, att2, b2, Wl3, bl3, Wr3, br3, We3, att3, b3,
           Wout, bout):
    src = edge_index[0]
    dst = edge_index[1]
    ew = _ew_transform(edge_attr.reshape(E))
    x_pad = jnp.zeros((P, x.shape[1]), jnp.float32).at[:N].set(x)
    zeros_block = jnp.zeros((P // 16, H), jnp.float32)

    h = _gat_layer_sc(x_pad, src, dst, ew, zeros_block, Wl1, bl1, Wr1, br1,
                      We1.reshape(H), att1, b1, relu=True)
    h = _gat_layer_sc(h, src, dst, ew, zeros_block, Wl2, bl2, Wr2, br2,
                      We2.reshape(H), att2, b2, relu=True)
    h = _gat_layer_sc(h, src, dst, ew, zeros_block, Wl3, bl3, Wr3, br3,
                      We3.reshape(H), att3, b3, relu=False)
    out = _outproj(h, Wout, bout)
    return out.reshape(P)[:N]
